# Initial kernel scaffold; baseline (speedup 1.0000x reference)
import jax
import jax.numpy as jnp
from jax.experimental import pallas as pl


def kernel(input_, target, bag):
    raise NotImplementedError("write your pallas kernel here")


# speedup vs baseline: 2.1005x; 2.1005x over previous
import functools

import jax
import jax.numpy as jnp
from jax import lax
from jax.experimental import pallas as pl
from jax.experimental.pallas import tpu as pltpu
from jax.experimental.pallas import tpu_sc as plsc

N = 320000
C = 128
M = 10000

NC = 2
NS = 16
NW = NC * NS

BPT = -(-((M + NW - 1) // NW) // 8) * 8
MP = BPT * NW
R = 256
NEG_INF = float("-inf")


def _segmax_body(input_hbm, bag_hbm, offs_hbm, out_hbm, in_v, bag_v, out_v,
                 offs_v, sem):
    wid = lax.axis_index("c") * NS + lax.axis_index("s")
    blo = wid * BPT

    pltpu.sync_copy(offs_hbm, offs_v)

    def init_row(d, _):
        for g in range(C // 16):
            out_v[d, pl.ds(g * 16, 16)] = jnp.full((16,), NEG_INF, jnp.float32)
        return 0
    lax.fori_loop(0, BPT, init_row, 0)

    ovec = offs_v[pl.ds(wid, 16)]
    rows_start = ovec[0]
    rows_end = ovec[1]
    start0 = jnp.minimum((rows_start // 8) * 8, N - R)
    nch = (rows_end - start0 + R - 1) // R

    def chunk_body(ci, _):
        rs = jnp.minimum(start0 + ci * R, N - R)
        pltpu.sync_copy(bag_hbm.at[pl.ds(rs, R)], bag_v)
        pltpu.sync_copy(input_hbm.at[pl.ds(rs, R)], in_v)

        def row16_body(k, _):
            base = k * 16
            vb = bag_v[pl.ds(base, 16)]
            for i in range(16):
                d = vb[i] - blo
                r = base + i

                @pl.when(jnp.logical_and(d >= 0, d < BPT))
                def _(d=d, r=r):
                    for g in range(C // 16):
                        sl = pl.ds(g * 16, 16)
                        out_v[d, sl] = jnp.maximum(out_v[d, sl], in_v[r, sl])
            return 0

        lax.fori_loop(0, R // 16, row16_body, 0)
        return 0

    lax.fori_loop(0, nch, chunk_body, 0)

    pltpu.sync_copy(out_v, out_hbm.at[pl.ds(blo, BPT)])


_segmax = functools.partial(
    pl.kernel,
    mesh=plsc.VectorSubcoreMesh(core_axis_name="c", subcore_axis_name="s"),
    out_type=jax.ShapeDtypeStruct((MP, C), jnp.float32),
    scratch_types=[
        pltpu.VMEM((R, C), jnp.float32),
        pltpu.VMEM((R,), jnp.int32),
        pltpu.VMEM((BPT, C), jnp.float32),
        pltpu.VMEM((48,), jnp.int32),
        pltpu.SemaphoreType.DMA,
    ],
)(_segmax_body)


def _ce_body(x_ref, t_ref, o_ref):
    x = x_ref[...]
    t = t_ref[...]
    m = jnp.max(x, axis=1, keepdims=True)
    e = jnp.exp(x - m)
    s = jnp.sum(e, axis=1, keepdims=True)
    lse = jnp.log(s) + m
    lanes = lax.broadcasted_iota(jnp.int32, x.shape, 1)
    picked = jnp.sum(jnp.where(lanes == t, x, 0.0), axis=1, keepdims=True)
    o_ref[0, 0] = jnp.sum(lse - picked) * (1.0 / M)


def _ce(x, t):
    return pl.pallas_call(
        _ce_body,
        out_shape=jax.ShapeDtypeStruct((1, 1), jnp.float32),
        out_specs=pl.BlockSpec(memory_space=pltpu.SMEM),
    )(x, t)


@jax.jit
def kernel(input_, target, bag):
    bag32 = bag.astype(jnp.int32)
    bounds = jnp.arange(NW + 1, dtype=jnp.int32) * BPT
    offs = jnp.searchsorted(bag32, bounds, side="left").astype(jnp.int32)
    offs_pad = jnp.zeros((48,), jnp.int32).at[: NW + 1].set(offs)
    seg = _segmax(input_, bag32, offs_pad)
    loss = _ce(seg[:M], target.astype(jnp.int32).reshape(M, 1))
    return loss[0, 0]


# speedup vs baseline: 4.6866x; 2.2311x over previous
import functools

import jax
import jax.numpy as jnp
from jax import lax
from jax.experimental import pallas as pl
from jax.experimental.pallas import tpu as pltpu
from jax.experimental.pallas import tpu_sc as plsc

N = 320000
C = 128
M = 10000

NC = 2
NS = 16
NW = NC * NS

BPT = -(-((M + NW - 1) // NW) // 8) * 8
MP = BPT * NW
R = 256
G = C // 16
NEG_INF = float("-inf")


def _segmax_body(input_hbm, bag_hbm, offs_hbm, out_hbm,
                 in_v0, in_v1, bag_v0, bag_v1, out_v, offs_v,
                 si0, si1, sb0, sb1):
    wid = lax.axis_index("c") * NS + lax.axis_index("s")
    blo = wid * BPT

    pltpu.sync_copy(offs_hbm, offs_v)

    def init_row(d, _):
        for g in range(G):
            out_v[d, pl.ds(g * 16, 16)] = jnp.full((16,), NEG_INF, jnp.float32)
        return 0
    lax.fori_loop(0, BPT, init_row, 0)

    ovec = offs_v[pl.ds(wid, 16)]
    rows_start = ovec[0]
    rows_end = ovec[1]
    start0 = jnp.minimum((rows_start // 8) * 8, N - R)
    nch = (rows_end - start0 + R - 1) // R

    def rs_of(ci):
        return jnp.minimum(start0 + ci * R, N - R)

    def start_chunk(ci, in_v, bag_v, si, sb):
        rs = rs_of(ci)
        pltpu.make_async_copy(input_hbm.at[pl.ds(rs, R)], in_v, si).start()
        pltpu.make_async_copy(bag_hbm.at[pl.ds(rs, R)], bag_v, sb).start()

    def wait_chunk(in_v, bag_v, si, sb):
        pltpu.make_async_copy(input_hbm.at[pl.ds(0, R)], in_v, si).wait()
        pltpu.make_async_copy(bag_hbm.at[pl.ds(0, R)], bag_v, sb).wait()

    def compute_chunk(in_v, bag_v, carry):
        def row16_body(k, carry):
            base = k * 16
            vb = bag_v[pl.ds(base, 16)]
            cur_d, acc = carry
            for i in range(16):
                d = vb[i] - blo
                change = d != cur_d

                @pl.when(jnp.logical_and(
                    change, jnp.logical_and(cur_d >= 0, cur_d < BPT)))
                def _(cur_d=cur_d, acc=acc):
                    for g in range(G):
                        sl = pl.ds(g * 16, 16)
                        out_v[cur_d, sl] = jnp.maximum(out_v[cur_d, sl], acc[g])

                r = base + i
                acc = tuple(
                    jnp.where(change,
                              in_v[r, pl.ds(g * 16, 16)],
                              jnp.maximum(acc[g], in_v[r, pl.ds(g * 16, 16)]))
                    for g in range(G))
                cur_d = d
            return cur_d, acc

        return lax.fori_loop(0, R // 16, row16_body, carry)

    acc0 = tuple(jnp.full((16,), NEG_INF, jnp.float32) for _ in range(G))
    carry0 = (jnp.int32(-1), acc0)

    @pl.when(nch > 0)
    def _():
        start_chunk(0, in_v0, bag_v0, si0, sb0)

    npairs = (nch + 1) // 2

    def pair_body(p, carry):
        ci1 = jnp.minimum(2 * p + 1, nch - 1)
        start_chunk(ci1, in_v1, bag_v1, si1, sb1)

        wait_chunk(in_v0, bag_v0, si0, sb0)
        carry = compute_chunk(in_v0, bag_v0, carry)

        @pl.when(p + 1 < npairs)
        def _():
            start_chunk(jnp.minimum(2 * p + 2, nch - 1), in_v0, bag_v0,
                        si0, sb0)

        wait_chunk(in_v1, bag_v1, si1, sb1)
        carry = compute_chunk(in_v1, bag_v1, carry)
        return carry

    cur_d, acc = lax.fori_loop(0, npairs, pair_body, carry0)

    @pl.when(jnp.logical_and(cur_d >= 0, cur_d < BPT))
    def _():
        for g in range(G):
            sl = pl.ds(g * 16, 16)
            out_v[cur_d, sl] = jnp.maximum(out_v[cur_d, sl], acc[g])

    pltpu.sync_copy(out_v, out_hbm.at[pl.ds(blo, BPT)])


_segmax = functools.partial(
    pl.kernel,
    mesh=plsc.VectorSubcoreMesh(core_axis_name="c", subcore_axis_name="s"),
    out_type=jax.ShapeDtypeStruct((MP, C), jnp.float32),
    scratch_types=[
        pltpu.VMEM((R, C), jnp.float32),
        pltpu.VMEM((R, C), jnp.float32),
        pltpu.VMEM((R,), jnp.int32),
        pltpu.VMEM((R,), jnp.int32),
        pltpu.VMEM((BPT, C), jnp.float32),
        pltpu.VMEM((48,), jnp.int32),
        pltpu.SemaphoreType.DMA,
        pltpu.SemaphoreType.DMA,
        pltpu.SemaphoreType.DMA,
        pltpu.SemaphoreType.DMA,
    ],
)(_segmax_body)


def _ce_body(x_ref, t_ref, o_ref):
    x = x_ref[...]
    t = t_ref[...]
    m = jnp.max(x, axis=1, keepdims=True)
    e = jnp.exp(x - m)
    s = jnp.sum(e, axis=1, keepdims=True)
    lse = jnp.log(s) + m
    lanes = lax.broadcasted_iota(jnp.int32, x.shape, 1)
    picked = jnp.sum(jnp.where(lanes == t, x, 0.0), axis=1, keepdims=True)
    o_ref[0, 0] = jnp.sum(lse - picked) * (1.0 / M)


def _ce(x, t):
    return pl.pallas_call(
        _ce_body,
        out_shape=jax.ShapeDtypeStruct((1, 1), jnp.float32),
        out_specs=pl.BlockSpec(memory_space=pltpu.SMEM),
    )(x, t)


@jax.jit
def kernel(input_, target, bag):
    bag32 = bag.astype(jnp.int32)
    bounds = jnp.arange(NW + 1, dtype=jnp.int32) * BPT
    offs = jnp.searchsorted(bag32, bounds, side="left").astype(jnp.int32)
    offs_pad = jnp.zeros((48,), jnp.int32).at[: NW + 1].set(offs)
    seg = _segmax(input_, bag32, offs_pad)
    loss = _ce(seg[:M], target.astype(jnp.int32).reshape(M, 1))
    return loss[0, 0]


# speedup vs baseline: 6.0216x; 1.2849x over previous
import functools

import jax
import jax.numpy as jnp
from jax import lax
from jax.experimental import pallas as pl
from jax.experimental.pallas import tpu as pltpu
from jax.experimental.pallas import tpu_sc as plsc

N = 320000
C = 128
M = 10000

NC = 2
NS = 16
NW = NC * NS

BPT = -(-((M + NW - 1) // NW) // 8) * 8
MP = BPT * NW
R = 256
G = C // 16
NEG_INF = float("-inf")


def _segmax_body(input_hbm, bag_hbm, offs_hbm, out_hbm,
                 in_v0, in_v1, bag_v0, bag_v1, out_v, offs_v,
                 si0, si1, sb0, sb1):
    wid = lax.axis_index("c") * NS + lax.axis_index("s")
    blo = wid * BPT
    lanes = lax.iota(jnp.int32, 16)

    pltpu.sync_copy(offs_hbm, offs_v)

    def init_row(d, _):
        for g in range(G):
            out_v[d, pl.ds(g * 16, 16)] = jnp.full((16,), NEG_INF, jnp.float32)
        return 0
    lax.fori_loop(0, BPT, init_row, 0)
    bag_v0[pl.ds(R, 16)] = jnp.full((16,), -1, jnp.int32)
    bag_v1[pl.ds(R, 16)] = jnp.full((16,), -1, jnp.int32)

    ovec = offs_v[pl.ds(wid, 16)]
    rows_start = ovec[0]
    rows_end = ovec[1]
    start0 = jnp.minimum((rows_start // 8) * 8, N - R)
    nch = (rows_end - start0 + R - 1) // R

    def rs_of(ci):
        return jnp.minimum(start0 + ci * R, N - R)

    def start_chunk(ci, in_v, bag_v, si, sb):
        rs = rs_of(ci)
        pltpu.make_async_copy(input_hbm.at[pl.ds(rs, R)], in_v, si).start()
        pltpu.make_async_copy(bag_hbm.at[pl.ds(rs, R)],
                              bag_v.at[pl.ds(0, R)], sb).start()

    def wait_chunk(in_v, bag_v, si, sb):
        pltpu.make_async_copy(input_hbm.at[pl.ds(0, R)], in_v, si).wait()
        pltpu.make_async_copy(bag_hbm.at[pl.ds(0, R)],
                              bag_v.at[pl.ds(0, R)], sb).wait()

    acc_init = tuple(jnp.full((16,), NEG_INF, jnp.float32) for _ in range(G))

    def flush(d, acc):
        @pl.when(jnp.logical_and(d >= 0, d < BPT))
        def _():
            for g in range(G):
                sl = pl.ds(g * 16, 16)
                out_v[d, sl] = jnp.maximum(out_v[d, sl], acc[g])

    def compute_chunk(in_v, bag_v):
        def group_body(k, _):
            base = k * 16
            vb = bag_v[pl.ds(base, 16)]
            b0 = vb[0]
            b15 = vb[15]

            ext = [vb[i] for i in range(16)]
            c = jnp.int32(1)
            sfx = jnp.int32(1)
            for i in range(1, 16):
                c = c + jnp.where(ext[i] == b0, 1, 0)
                sfx = sfx + jnp.where(ext[i - 1] == b15, 1, 0)
            c2 = jnp.maximum(16 - sfx, c)

            def row_body(i, acc):
                r = base + i
                return tuple(
                    jnp.maximum(acc[g], in_v[r, pl.ds(g * 16, 16)])
                    for g in range(G))

            flush(b0 - blo, lax.fori_loop(0, c, row_body, acc_init))
            flush(b15 - blo, lax.fori_loop(c2, 16, row_body, acc_init))

            @pl.when(c2 > c)
            def _():
                for i in range(16):
                    d_i = ext[i] - blo
                    ok = jnp.logical_and(
                        jnp.logical_and(i >= c, i < c2),
                        jnp.logical_and(d_i >= 0, d_i < BPT))

                    @pl.when(ok)
                    def _(d_i=d_i, i=i):
                        for g in range(G):
                            sl = pl.ds(g * 16, 16)
                            out_v[d_i, sl] = jnp.maximum(
                                out_v[d_i, sl], in_v[base + i, sl])
            return 0

        lax.fori_loop(0, R // 16, group_body, 0)

    @pl.when(nch > 0)
    def _():
        start_chunk(0, in_v0, bag_v0, si0, sb0)

    npairs = (nch + 1) // 2

    def pair_body(p, _):
        ci1 = jnp.minimum(2 * p + 1, nch - 1)
        start_chunk(ci1, in_v1, bag_v1, si1, sb1)

        wait_chunk(in_v0, bag_v0, si0, sb0)
        compute_chunk(in_v0, bag_v0)

        @pl.when(p + 1 < npairs)
        def _():
            start_chunk(jnp.minimum(2 * p + 2, nch - 1), in_v0, bag_v0,
                        si0, sb0)

        wait_chunk(in_v1, bag_v1, si1, sb1)
        compute_chunk(in_v1, bag_v1)
        return 0

    lax.fori_loop(0, npairs, pair_body, 0)

    pltpu.sync_copy(out_v, out_hbm.at[pl.ds(blo, BPT)])


_segmax = functools.partial(
    pl.kernel,
    mesh=plsc.VectorSubcoreMesh(core_axis_name="c", subcore_axis_name="s"),
    out_type=jax.ShapeDtypeStruct((MP, C), jnp.float32),
    scratch_types=[
        pltpu.VMEM((R, C), jnp.float32),
        pltpu.VMEM((R, C), jnp.float32),
        pltpu.VMEM((R + 16,), jnp.int32),
        pltpu.VMEM((R + 16,), jnp.int32),
        pltpu.VMEM((BPT, C), jnp.float32),
        pltpu.VMEM((48,), jnp.int32),
        pltpu.SemaphoreType.DMA,
        pltpu.SemaphoreType.DMA,
        pltpu.SemaphoreType.DMA,
        pltpu.SemaphoreType.DMA,
    ],
)(_segmax_body)


def _ce_body(x_ref, t_ref, o_ref):
    x = x_ref[...]
    t = t_ref[...]
    m = jnp.max(x, axis=1, keepdims=True)
    e = jnp.exp(x - m)
    s = jnp.sum(e, axis=1, keepdims=True)
    lse = jnp.log(s) + m
    lanes = lax.broadcasted_iota(jnp.int32, x.shape, 1)
    picked = jnp.sum(jnp.where(lanes == t, x, 0.0), axis=1, keepdims=True)
    rows = lax.broadcasted_iota(jnp.int32, (MP, 1), 0)
    nll = jnp.where(rows < M, lse - picked, 0.0)
    o_ref[0, 0] = jnp.sum(nll) * (1.0 / M)


def _ce(x, t):
    return pl.pallas_call(
        _ce_body,
        out_shape=jax.ShapeDtypeStruct((1, 1), jnp.float32),
        out_specs=pl.BlockSpec(memory_space=pltpu.SMEM),
    )(x, t)


@jax.jit
def kernel(input_, target, bag):
    bag32 = bag.astype(jnp.int32)
    bounds = jnp.arange(NW + 1, dtype=jnp.int32) * BPT
    offs = jnp.searchsorted(bag32, bounds, side="left").astype(jnp.int32)
    offs_pad = jnp.zeros((48,), jnp.int32).at[: NW + 1].set(offs)
    seg = _segmax(input_, bag32, offs_pad)
    t_pad = jnp.zeros((MP, 1), jnp.int32).at[:M, 0].set(
        target.astype(jnp.int32))
    return _ce(seg, t_pad)[0, 0]
